# baseline (speedup 1.0000x reference)
import functools

import jax
import jax.numpy as jnp
from jax import lax
from jax.experimental import pallas as pl
from jax.experimental.pallas import tpu as pltpu
from jax.experimental.pallas import tpu_sc as plsc

D_MODEL = 768
N_HEADS = 12
D_HEAD = 64
N_CLUSTERS = 8
B_SZ = 2
SEQ = 2048
BS = 256
BQ = 256
BK = 256
BIAS = 1024.0
CHUNK = 128
I32_PER_ROW = D_MODEL // 2


def _route_kernel(q_ref, k_ref, wr_ref, qa_ref, ka_ref):
    def assign(blk):
        logits = jnp.dot(blk, wr_ref[...], preferred_element_type=jnp.float32)
        idx = lax.broadcasted_iota(jnp.int32, logits.shape, 1)
        mx = jnp.max(logits, axis=-1, keepdims=True)
        am = jnp.min(jnp.where(logits == mx, idx, N_CLUSTERS), axis=-1,
                     keepdims=True)
        return am.astype(jnp.float32)

    qa_ref[0] = assign(q_ref[0])
    ka_ref[0] = assign(k_ref[0])


def _proj_kernel(k_ref, v_ref, wk_ref, wv_ref, kp_ref, vp_ref):
    kblk = k_ref[0]
    vblk = v_ref[0]
    for h in range(N_HEADS):
        kp_ref[0, h] = jnp.dot(kblk, wk_ref[h],
                               preferred_element_type=jnp.float32
                               ).astype(jnp.bfloat16)
        vp_ref[0, h] = jnp.dot(vblk, wv_ref[h],
                               preferred_element_type=jnp.float32
                               ).astype(jnp.bfloat16)


def _sort_kernel(qa_ref, ka_ref, pq_ref, pk_ref, offs_ref):
    S = SEQ
    M = N_CLUSTERS
    row_i = lax.broadcasted_iota(jnp.int32, (S, 1), 0)
    col_j = lax.broadcasted_iota(jnp.int32, (1, S), 1)
    tril = (col_j < row_i).astype(jnp.bfloat16)
    ioc = lax.broadcasted_iota(jnp.int32, (1, M), 1).astype(jnp.float32)
    tri8 = (lax.broadcasted_iota(jnp.int32, (M, 1), 0)
            < lax.broadcasted_iota(jnp.int32, (1, M), 1)).astype(jnp.float32)
    pcol = lax.broadcasted_iota(jnp.int32, (1, S), 1).astype(jnp.float32)
    idx_i = lax.broadcasted_iota(jnp.int32, (S, 1), 0)
    idx_hi = (idx_i // 8).astype(jnp.float32)
    idx_lo = (idx_i - (idx_i // 8) * 8).astype(jnp.float32)

    def side(asg):
        oh = (asg == ioc).astype(jnp.bfloat16)
        ranks = jnp.dot(tril, oh, preferred_element_type=jnp.float32)
        ohf = oh.astype(jnp.float32)
        counts = jnp.sum(ohf, axis=0, keepdims=True)
        offs = jnp.dot(counts, tri8, preferred_element_type=jnp.float32,
                       precision=lax.Precision.HIGHEST)
        pos = jnp.sum((ranks + offs) * ohf, axis=1, keepdims=True)
        ind = (pos == pcol).astype(jnp.bfloat16)
        perm = (8.0 * lax.dot_general(ind, idx_hi.astype(jnp.bfloat16),
                                      (((0,), (0,)), ((), ())),
                                      preferred_element_type=jnp.float32)
                + lax.dot_general(ind, idx_lo.astype(jnp.bfloat16),
                                  (((0,), (0,)), ((), ())),
                                  preferred_element_type=jnp.float32))
        return perm, offs

    pq, oq = side(qa_ref[0])
    pk, ok = side(ka_ref[0])
    pq_ref[0] = pq
    pk_ref[0] = pk
    pad = jnp.full((1, 16 - M), float(S), dtype=jnp.float32)
    offs_ref[0] = jnp.concatenate(
        [jnp.concatenate([oq, pad], axis=1),
         jnp.concatenate([ok, pad], axis=1)], axis=0)


def _attn_kernel(qs_ref, wq_ref, kps_ref, vps_ref, offs_ref, wo_ref,
                 out_ref):
    b = pl.program_id(0)
    i = pl.program_id(1)
    h = pl.program_id(2)
    S = SEQ
    M = N_CLUSTERS
    qlo = i * BQ

    def scan_offs(c, carry):
        c_lo, c_hi = carry
        qc = offs_ref[b, 0, c]
        c_lo = c_lo + jnp.where(qc <= qlo, 1, 0)
        c_hi = c_hi + jnp.where(qc < qlo + BQ, 1, 0)
        return c_lo, c_hi

    c_lo, c_hi = lax.fori_loop(1, M, scan_offs, (0, 0))
    kstart = offs_ref[b, 1, c_lo]
    kend = offs_ref[b, 1, c_hi + 1]
    t0 = kstart // BK
    t1 = (kend + BK - 1) // BK

    qpos = qlo + lax.broadcasted_iota(jnp.int32, (BQ, 1), 0)

    def cl_of(pos, kside):
        def body(c, acc):
            return acc + jnp.where(pos >= offs_ref[b, kside, c], 1, 0)
        return lax.fori_loop(1, M, body, jnp.zeros(pos.shape, jnp.int32))

    qcl = cl_of(qpos, 0)
    ioc_row = lax.broadcasted_iota(jnp.int32, (1, M), 1)
    qoh = (qcl == ioc_row).astype(jnp.bfloat16) * jnp.bfloat16(BIAS)

    qh = jnp.dot(qs_ref[0], wq_ref[0], preferred_element_type=jnp.float32)
    qa = jnp.concatenate([(qh * 0.125).astype(jnp.bfloat16), qoh], axis=1)

    def tile(t, carry):
        ctx_un, denom = carry
        off = t * BK
        kh = kps_ref[0, h, pl.ds(off, BK), :]
        vh = vps_ref[0, h, pl.ds(off, BK), :]
        kpos = off + lax.broadcasted_iota(jnp.int32, (BK, 1), 0)
        kcl = cl_of(kpos, 1)
        koh = (kcl == ioc_row).astype(jnp.bfloat16)
        ka = jnp.concatenate([kh, koh], axis=1)
        s = lax.dot_general(qa, ka, (((1,), (1,)), ((), ())),
                            preferred_element_type=jnp.float32)
        e = jnp.exp(s - BIAS)
        denom = denom + jnp.sum(e, axis=-1, keepdims=True)
        ctx_un = ctx_un + jnp.dot(e.astype(jnp.bfloat16), vh,
                                  preferred_element_type=jnp.float32)
        return ctx_un, denom

    ctx_un, denom = lax.fori_loop(
        t0, t1, tile,
        (jnp.zeros((BQ, D_HEAD), jnp.float32), jnp.zeros((BQ, 1),
                                                         jnp.float32)))
    r = jnp.where(denom > 0.0, 1.0 / denom, jnp.float32(0.0))
    ctx = ctx_un * r
    contrib = jnp.dot(ctx.astype(jnp.bfloat16), wo_ref[...],
                      preferred_element_type=jnp.float32)

    @pl.when(h == 0)
    def _():
        out_ref[0] = contrib

    @pl.when(h != 0)
    def _():
        out_ref[0] += contrib


def _chunk_affine(src_ref, dst_ref, scale, offset):
    for j in range(CHUNK // 16):
        sl = pl.ds(j * 16, 16)
        dst_ref[sl] = src_ref[sl] * scale + offset


def _sc_gather(qflat, kflat, vflat, permq, permk):
    B, S = B_SZ, SEQ
    W = I32_PER_ROW
    mesh = plsc.VectorSubcoreMesh(core_axis_name="c", subcore_axis_name="s")

    @functools.partial(
        pl.kernel, mesh=mesh,
        out_type=[
            jax.ShapeDtypeStruct((B * S, W), jnp.int32),
            jax.ShapeDtypeStruct((B * S, W), jnp.int32),
            jax.ShapeDtypeStruct((B * S, W), jnp.int32),
        ],
        scratch_types=[
            pltpu.VMEM((CHUNK,), jnp.int32),
            pltpu.VMEM((CHUNK,), jnp.int32),
            pltpu.VMEM((CHUNK, W), jnp.int32),
            pltpu.SemaphoreType.DMA,
        ],
    )
    def gather_kernel(q_hbm, k_hbm, v_hbm, pq_hbm, pk_hbm,
                      qs_hbm, ks_hbm, vs_hbm,
                      idx_v, idx2_v, rows_v, sem):
        b = lax.axis_index("c")
        w = lax.axis_index("s")
        base = w * CHUNK
        pltpu.sync_copy(pq_hbm.at[pl.ds(b * S + base, CHUNK)], idx_v)
        _chunk_affine(idx_v, idx2_v, 1, b * S)
        pltpu.async_copy(q_hbm.at[idx2_v], rows_v, sem).wait()
        pltpu.sync_copy(rows_v, qs_hbm.at[pl.ds(b * S + base, CHUNK)])
        pltpu.sync_copy(pk_hbm.at[pl.ds(b * S + base, CHUNK)], idx_v)
        _chunk_affine(idx_v, idx2_v, 1, b * S)
        pltpu.async_copy(k_hbm.at[idx2_v], rows_v, sem).wait()
        pltpu.sync_copy(rows_v, ks_hbm.at[pl.ds(b * S + base, CHUNK)])
        pltpu.async_copy(v_hbm.at[idx2_v], rows_v, sem).wait()
        pltpu.sync_copy(rows_v, vs_hbm.at[pl.ds(b * S + base, CHUNK)])

    return gather_kernel(qflat, kflat, vflat, permq, permk)


def _sc_scatter(outs_flat, permq):
    B, S = B_SZ, SEQ
    mesh = plsc.VectorSubcoreMesh(core_axis_name="c", subcore_axis_name="s")

    @functools.partial(
        pl.kernel, mesh=mesh,
        out_type=jax.ShapeDtypeStruct((B * S, D_MODEL), jnp.float32),
        scratch_types=[
            pltpu.VMEM((CHUNK,), jnp.int32),
            pltpu.VMEM((CHUNK,), jnp.int32),
            pltpu.VMEM((CHUNK, D_MODEL), jnp.float32),
            pltpu.SemaphoreType.DMA,
        ],
    )
    def scatter_kernel(src_hbm, pq_hbm, out_hbm, idx_v, idx2_v, rows_v,
                       sem):
        b = lax.axis_index("c")
        w = lax.axis_index("s")
        base = w * CHUNK
        pltpu.sync_copy(pq_hbm.at[pl.ds(b * S + base, CHUNK)], idx_v)
        _chunk_affine(idx_v, idx2_v, 1, b * S)
        pltpu.sync_copy(src_hbm.at[pl.ds(b * S + base, CHUNK)], rows_v)
        pltpu.async_copy(rows_v, out_hbm.at[idx2_v], sem).wait()

    return scatter_kernel(outs_flat, permq)


def _bitcast_to_i32(x16):
    shp = x16.shape
    return lax.bitcast_convert_type(
        x16.reshape(shp[:-1] + (shp[-1] // 2, 2)), jnp.int32)


def kernel(q, k, v, Wq, Wk, Wv, Wo, Wr):
    B, Sq, D = q.shape
    Sk = k.shape[1]
    H, dh, M = N_HEADS, D_HEAD, N_CLUSTERS

    WqT = Wq.reshape(D, H, dh).transpose(1, 0, 2).astype(jnp.bfloat16)
    WkT = Wk.reshape(D, H, dh).transpose(1, 0, 2).astype(jnp.bfloat16)
    WvT = Wv.reshape(D, H, dh).transpose(1, 0, 2).astype(jnp.bfloat16)
    Wo16 = Wo.astype(jnp.bfloat16)

    nb = Sk // BS
    qasg, kasg = pl.pallas_call(
        _route_kernel,
        grid=(B, nb),
        in_specs=[
            pl.BlockSpec((1, BS, D), lambda b, i: (b, i, 0)),
            pl.BlockSpec((1, BS, D), lambda b, i: (b, i, 0)),
            pl.BlockSpec((D, M), lambda b, i: (0, 0)),
        ],
        out_specs=[
            pl.BlockSpec((1, BS, 1), lambda b, i: (b, i, 0)),
            pl.BlockSpec((1, BS, 1), lambda b, i: (b, i, 0)),
        ],
        out_shape=[
            jax.ShapeDtypeStruct((B, Sq, 1), jnp.float32),
            jax.ShapeDtypeStruct((B, Sk, 1), jnp.float32),
        ],
    )(q, k, Wr)

    permq_f, permk_f, offs_f = pl.pallas_call(
        _sort_kernel,
        grid=(B,),
        in_specs=[
            pl.BlockSpec((1, Sq, 1), lambda b: (b, 0, 0)),
            pl.BlockSpec((1, Sk, 1), lambda b: (b, 0, 0)),
        ],
        out_specs=[
            pl.BlockSpec((1, Sq, 1), lambda b: (b, 0, 0)),
            pl.BlockSpec((1, Sk, 1), lambda b: (b, 0, 0)),
            pl.BlockSpec((1, 2, 16), lambda b: (b, 0, 0)),
        ],
        out_shape=[
            jax.ShapeDtypeStruct((B, Sq, 1), jnp.float32),
            jax.ShapeDtypeStruct((B, Sk, 1), jnp.float32),
            jax.ShapeDtypeStruct((B, 2, 16), jnp.float32),
        ],
    )(qasg, kasg)

    permq = permq_f.reshape(B * Sq).astype(jnp.int32)
    permk = permk_f.reshape(B * Sk).astype(jnp.int32)
    offs = offs_f.astype(jnp.int32)

    q16 = q.astype(jnp.bfloat16)
    k16 = k.astype(jnp.bfloat16)
    v16 = v.astype(jnp.bfloat16)
    qflat = _bitcast_to_i32(q16.reshape(B * Sq, D))
    kflat = _bitcast_to_i32(k16.reshape(B * Sk, D))
    vflat = _bitcast_to_i32(v16.reshape(B * Sk, D))

    qs_i, ks_i, vs_i = _sc_gather(qflat, kflat, vflat, permq, permk)

    qs = lax.bitcast_convert_type(qs_i, jnp.bfloat16).reshape(B, Sq, D)
    ks = lax.bitcast_convert_type(ks_i, jnp.bfloat16).reshape(B, Sk, D)
    vs = lax.bitcast_convert_type(vs_i, jnp.bfloat16).reshape(B, Sk, D)

    kps, vps = pl.pallas_call(
        _proj_kernel,
        grid=(B, nb),
        in_specs=[
            pl.BlockSpec((1, BS, D), lambda b, i: (b, i, 0)),
            pl.BlockSpec((1, BS, D), lambda b, i: (b, i, 0)),
            pl.BlockSpec((H, D, dh), lambda b, i: (0, 0, 0)),
            pl.BlockSpec((H, D, dh), lambda b, i: (0, 0, 0)),
        ],
        out_specs=[
            pl.BlockSpec((1, H, BS, dh), lambda b, i: (b, 0, i, 0)),
            pl.BlockSpec((1, H, BS, dh), lambda b, i: (b, 0, i, 0)),
        ],
        out_shape=[
            jax.ShapeDtypeStruct((B, H, Sk, dh), jnp.bfloat16),
            jax.ShapeDtypeStruct((B, H, Sk, dh), jnp.bfloat16),
        ],
    )(ks, vs, WkT, WvT)

    nq = Sq // BQ
    outs = pl.pallas_call(
        _attn_kernel,
        grid=(B, nq, H),
        in_specs=[
            pl.BlockSpec((1, BQ, D), lambda b, i, h: (b, i, 0)),
            pl.BlockSpec((1, D, dh), lambda b, i, h: (h, 0, 0)),
            pl.BlockSpec((1, H, Sk, dh), lambda b, i, h: (b, 0, 0, 0)),
            pl.BlockSpec((1, H, Sk, dh), lambda b, i, h: (b, 0, 0, 0)),
            pl.BlockSpec(memory_space=pltpu.SMEM),
            pl.BlockSpec((dh, D), lambda b, i, h: (h, 0)),
        ],
        out_specs=pl.BlockSpec((1, BQ, D), lambda b, i, h: (b, i, 0)),
        out_shape=jax.ShapeDtypeStruct((B, Sq, D), jnp.float32),
    )(qs, WqT, kps, vps, offs, Wo16)

    out = _sc_scatter(outs.reshape(B * Sq, D), permq).reshape(B, Sq, D)
    return out


# speedup vs baseline: 3.2008x; 3.2008x over previous
import jax
import jax.numpy as jnp
from jax.experimental import pallas as pl
from jax.experimental.pallas import tpu as pltpu

D_MODEL = 768
N_HEADS = 12
D_HEAD = 64
N_CLUSTERS = 8
BS = 256
BQ = 512
BIAS = 1024.0


def _onehot_argmax(logits):
    idx = jax.lax.broadcasted_iota(jnp.int32, logits.shape, 1)
    mx = jnp.max(logits, axis=-1, keepdims=True)
    am = jnp.min(jnp.where(logits == mx, idx, logits.shape[-1]), axis=-1,
                 keepdims=True)
    return (idx == am).astype(jnp.float32)


def _proj_route_kernel(q_ref, k_ref, v_ref, wk_ref, wv_ref, wr_ref,
                       kp_ref, vp_ref, qoh_ref, koh_ref):
    qblk = q_ref[0]
    kblk = k_ref[0]
    vblk = v_ref[0]
    kb16 = kblk.astype(jnp.bfloat16)
    vb16 = vblk.astype(jnp.bfloat16)
    for h in range(N_HEADS):
        kp_ref[0, h] = jnp.dot(kb16, wk_ref[h],
                               preferred_element_type=jnp.float32
                               ).astype(jnp.bfloat16)
        vp_ref[0, h] = jnp.dot(vb16, wv_ref[h],
                               preferred_element_type=jnp.float32
                               ).astype(jnp.bfloat16)
    qoh_ref[0] = _onehot_argmax(
        jnp.dot(qblk, wr_ref[...], preferred_element_type=jnp.float32)
    ).astype(jnp.bfloat16)
    koh_ref[0] = _onehot_argmax(
        jnp.dot(kblk, wr_ref[...], preferred_element_type=jnp.float32)
    ).astype(jnp.bfloat16)


def _attn_kernel(q_ref, wq_ref, kp_ref, vp_ref, qoh_ref, koh_ref, wo_ref,
                 out_ref):
    h = pl.program_id(2)
    qh = jnp.dot(q_ref[0].astype(jnp.bfloat16), wq_ref[0],
                 preferred_element_type=jnp.float32)
    qa = jnp.concatenate(
        [(qh * 0.125).astype(jnp.bfloat16), qoh_ref[0] * BIAS], axis=1)
    kh = kp_ref[0, h]
    vh = vp_ref[0, h]
    ka = jnp.concatenate([kh, koh_ref[0]], axis=1)
    s = jax.lax.dot_general(qa, ka, (((1,), (1,)), ((), ())),
                            preferred_element_type=jnp.float32)
    e = jnp.exp(s - BIAS)
    denom = jnp.sum(e, axis=-1, keepdims=True)
    ctx_un = jnp.dot(e.astype(jnp.bfloat16), vh,
                     preferred_element_type=jnp.float32)
    r = jnp.where(denom > 0.0, 1.0 / denom, jnp.float32(0.0))
    ctx = ctx_un * r
    contrib = jnp.dot(ctx.astype(jnp.bfloat16), wo_ref[...],
                      preferred_element_type=jnp.float32)

    @pl.when(h == 0)
    def _():
        out_ref[0] = contrib

    @pl.when(h != 0)
    def _():
        out_ref[0] += contrib


def kernel(q, k, v, Wq, Wk, Wv, Wo, Wr):
    B, Sq, D = q.shape
    Sk = k.shape[1]
    H, dh, M = N_HEADS, D_HEAD, N_CLUSTERS

    WqT = Wq.reshape(D, H, dh).transpose(1, 0, 2).astype(jnp.bfloat16)
    WkT = Wk.reshape(D, H, dh).transpose(1, 0, 2).astype(jnp.bfloat16)
    WvT = Wv.reshape(D, H, dh).transpose(1, 0, 2).astype(jnp.bfloat16)
    Wo16 = Wo.astype(jnp.bfloat16)

    nb = Sk // BS
    kp, vp, qoh, koh = pl.pallas_call(
        _proj_route_kernel,
        grid=(B, nb),
        compiler_params=pltpu.CompilerParams(
            dimension_semantics=("parallel", "parallel")),
        in_specs=[
            pl.BlockSpec((1, BS, D), lambda b, i: (b, i, 0)),
            pl.BlockSpec((1, BS, D), lambda b, i: (b, i, 0)),
            pl.BlockSpec((1, BS, D), lambda b, i: (b, i, 0)),
            pl.BlockSpec((H, D, dh), lambda b, i: (0, 0, 0)),
            pl.BlockSpec((H, D, dh), lambda b, i: (0, 0, 0)),
            pl.BlockSpec((D, M), lambda b, i: (0, 0)),
        ],
        out_specs=[
            pl.BlockSpec((1, H, BS, dh), lambda b, i: (b, 0, i, 0)),
            pl.BlockSpec((1, H, BS, dh), lambda b, i: (b, 0, i, 0)),
            pl.BlockSpec((1, BS, M), lambda b, i: (b, i, 0)),
            pl.BlockSpec((1, BS, M), lambda b, i: (b, i, 0)),
        ],
        out_shape=[
            jax.ShapeDtypeStruct((B, H, Sk, dh), jnp.bfloat16),
            jax.ShapeDtypeStruct((B, H, Sk, dh), jnp.bfloat16),
            jax.ShapeDtypeStruct((B, Sq, M), jnp.bfloat16),
            jax.ShapeDtypeStruct((B, Sk, M), jnp.bfloat16),
        ],
    )(q, k, v, WkT, WvT, Wr)

    nq = Sq // BQ
    out = pl.pallas_call(
        _attn_kernel,
        grid=(B, nq, H),
        compiler_params=pltpu.CompilerParams(
            dimension_semantics=("parallel", "parallel", "arbitrary")),
        in_specs=[
            pl.BlockSpec((1, BQ, D), lambda b, i, h: (b, i, 0)),
            pl.BlockSpec((1, D, dh), lambda b, i, h: (h, 0, 0)),
            pl.BlockSpec((1, H, Sk, dh), lambda b, i, h: (b, 0, 0, 0)),
            pl.BlockSpec((1, H, Sk, dh), lambda b, i, h: (b, 0, 0, 0)),
            pl.BlockSpec((1, BQ, M), lambda b, i, h: (b, i, 0)),
            pl.BlockSpec((1, Sk, M), lambda b, i, h: (b, 0, 0)),
            pl.BlockSpec((dh, D), lambda b, i, h: (h, 0)),
        ],
        out_specs=pl.BlockSpec((1, BQ, D), lambda b, i, h: (b, i, 0)),
        out_shape=jax.ShapeDtypeStruct((B, Sq, D), jnp.float32),
    )(q, WqT, kp, vp, qoh, koh, Wo16)
    return out
